# baseline (device time: 9302 ns/iter reference)
import jax
import jax.numpy as jnp
from jax import lax
from jax.experimental import pallas as pl
from jax.experimental.pallas import tpu as pltpu

N_Y = 4


def kernel(x, dy, gamma):
    m, d = x.shape

    def body(x_ref, dy_ref, out_ref, comm_ref, send_sems, recv_sems):
        mx = lax.axis_index("x")
        my = lax.axis_index("y")
        mz = lax.axis_index("z")

        xv = x_ref[:, :]
        dyv = dy_ref[:, :]
        mu = jnp.mean(xv, axis=1, keepdims=True)
        xc = xv - mu
        var = jnp.mean(xc * xc, axis=1, keepdims=True)
        rstd = lax.rsqrt(var + 1e-5)
        dgamma = jnp.sum(dyv * (xc * rstd), axis=0)
        dbeta = jnp.sum(dyv, axis=0)
        comm_ref[my] = jnp.concatenate([dgamma[None, :], dbeta[None, :]], axis=0)

        barrier = pltpu.get_barrier_semaphore()
        for dd in range(1, N_Y):
            py = (my + dd) % N_Y
            pl.semaphore_signal(
                barrier, inc=1,
                device_id=(mx, py, mz), device_id_type=pl.DeviceIdType.MESH,
            )
        pl.semaphore_wait(barrier, N_Y - 1)

        sends = []
        for dd in range(1, N_Y):
            py = (my + dd) % N_Y
            rdma = pltpu.make_async_remote_copy(
                src_ref=comm_ref.at[my],
                dst_ref=comm_ref.at[my],
                send_sem=send_sems.at[dd - 1],
                recv_sem=recv_sems.at[my],
                device_id=(mx, py, mz),
                device_id_type=pl.DeviceIdType.MESH,
            )
            rdma.start()
            sends.append(rdma)

        for dd in range(1, N_Y):
            sy = (my + dd) % N_Y
            recv = pltpu.make_async_remote_copy(
                src_ref=comm_ref.at[sy],
                dst_ref=comm_ref.at[sy],
                send_sem=send_sems.at[dd - 1],
                recv_sem=recv_sems.at[sy],
                device_id=(mx, my, mz),
                device_id_type=pl.DeviceIdType.MESH,
            )
            recv.wait_recv()

        out_ref[:, :] = comm_ref[0] + comm_ref[1] + comm_ref[2] + comm_ref[3]

        for s in sends:
            s.wait_send()

    return pl.pallas_call(
        body,
        out_shape=jax.ShapeDtypeStruct((2, d), jnp.float32),
        in_specs=[
            pl.BlockSpec(memory_space=pltpu.VMEM),
            pl.BlockSpec(memory_space=pltpu.VMEM),
        ],
        out_specs=pl.BlockSpec(memory_space=pltpu.VMEM),
        scratch_shapes=[
            pltpu.VMEM((N_Y, 2, d), jnp.float32),
            pltpu.SemaphoreType.DMA((N_Y - 1,)),
            pltpu.SemaphoreType.DMA((N_Y,)),
        ],
        compiler_params=pltpu.CompilerParams(collective_id=0),
    )(x, dy)


# device time: 8944 ns/iter; 1.0400x vs baseline; 1.0400x over previous
import jax
import jax.numpy as jnp
from jax import lax
from jax.experimental import pallas as pl
from jax.experimental.pallas import tpu as pltpu

N_Y = 4


def kernel(x, dy, gamma):
    m, d = x.shape

    def body(x_ref, dy_ref, out_ref, comm_ref, send_sems, recv_sems):
        mx = lax.axis_index("x")
        my = lax.axis_index("y")
        mz = lax.axis_index("z")

        barrier = pltpu.get_barrier_semaphore()
        for dd in range(1, N_Y):
            py = (my + dd) % N_Y
            pl.semaphore_signal(
                barrier, inc=1,
                device_id=(mx, py, mz), device_id_type=pl.DeviceIdType.MESH,
            )

        xv = x_ref[:, :]
        dyv = dy_ref[:, :]
        mu = jnp.mean(xv, axis=1, keepdims=True)
        xc = xv - mu
        var = jnp.mean(xc * xc, axis=1, keepdims=True)
        rstd = lax.rsqrt(var + 1e-5)
        dgamma = jnp.sum(dyv * (xc * rstd), axis=0)
        dbeta = jnp.sum(dyv, axis=0)
        comm_ref[my] = jnp.concatenate([dgamma[None, :], dbeta[None, :]], axis=0)

        pl.semaphore_wait(barrier, N_Y - 1)

        sends = []
        for dd in range(1, N_Y):
            py = (my + dd) % N_Y
            rdma = pltpu.make_async_remote_copy(
                src_ref=comm_ref.at[my],
                dst_ref=comm_ref.at[my],
                send_sem=send_sems.at[dd - 1],
                recv_sem=recv_sems.at[my],
                device_id=(mx, py, mz),
                device_id_type=pl.DeviceIdType.MESH,
            )
            rdma.start()
            sends.append(rdma)

        for dd in range(1, N_Y):
            sy = (my + dd) % N_Y
            recv = pltpu.make_async_remote_copy(
                src_ref=comm_ref.at[sy],
                dst_ref=comm_ref.at[sy],
                send_sem=send_sems.at[dd - 1],
                recv_sem=recv_sems.at[sy],
                device_id=(mx, my, mz),
                device_id_type=pl.DeviceIdType.MESH,
            )
            recv.wait_recv()

        out_ref[:, :] = comm_ref[0] + comm_ref[1] + comm_ref[2] + comm_ref[3]

        for s in sends:
            s.wait_send()

    return pl.pallas_call(
        body,
        out_shape=jax.ShapeDtypeStruct((2, d), jnp.float32),
        in_specs=[
            pl.BlockSpec(memory_space=pltpu.VMEM),
            pl.BlockSpec(memory_space=pltpu.VMEM),
        ],
        out_specs=pl.BlockSpec(memory_space=pltpu.VMEM),
        scratch_shapes=[
            pltpu.VMEM((N_Y, 2, d), jnp.float32),
            pltpu.SemaphoreType.DMA((N_Y - 1,)),
            pltpu.SemaphoreType.DMA((N_Y,)),
        ],
        compiler_params=pltpu.CompilerParams(collective_id=0),
    )(x, dy)
